# baseline (speedup 1.0000x reference)
import jax
import jax.numpy as jnp
from jax.experimental import pallas as pl
from jax.experimental.pallas import tpu as pltpu

_NF = 5000
_NC = 80
_FH = 64
_FW = 64
_KC = 1000
_PAD = 1024
_THR = 0.65
_KD = 100
_BLK = 200
_CH = 128
_NR = 1024


def _prep_body(seg_ref, cls_ref, scores_ref, boxes_ref, rowmax_ref):
    m = seg_ref[...] > 0.0
    s_i = jax.lax.broadcasted_iota(jnp.int32, m.shape, 1)
    l_i = jax.lax.broadcasted_iota(jnp.int32, m.shape, 2)
    h = 2 * s_i + (l_i >= 64).astype(jnp.int32)
    w = l_i & 63
    x1 = jnp.min(jnp.min(jnp.where(m, w, _FW), axis=2), axis=1, keepdims=True)
    x2 = jnp.max(jnp.max(jnp.where(m, w + 1, 0), axis=2), axis=1, keepdims=True)
    y1 = jnp.min(jnp.min(jnp.where(m, h, _FH), axis=2), axis=1, keepdims=True)
    y2 = jnp.max(jnp.max(jnp.where(m, h + 1, 0), axis=2), axis=1, keepdims=True)
    boxes_ref[...] = jnp.concatenate([x1, y1, x2, y2], axis=1).astype(jnp.float32)
    non_empty = jnp.any(jnp.any(m, axis=2), axis=1, keepdims=True)
    sc = jnp.where(non_empty, jax.nn.sigmoid(cls_ref[...]), -1.0)
    scores_ref[...] = sc
    rowmax_ref[...] = jnp.max(sc, axis=1, keepdims=True)


def _nms_body(brow_ref, bcol_ref, labr_ref, labc_ref, sc_ref, out_ref,
              sup_ref, supd_ref):
    x1 = brow_ref[0:1, :]
    y1 = brow_ref[1:2, :]
    x2 = brow_ref[2:3, :]
    y2 = brow_ref[3:4, :]
    lanes = jax.lax.broadcasted_iota(jnp.int32, (1, _PAD), 1)
    valid = lanes < _KC
    m4 = jnp.maximum(jnp.maximum(x1, x2), jnp.maximum(y1, y2))
    maxc = jnp.max(jnp.where(valid, m4, -1.0))
    scale = maxc + 1.0
    offr = labr_ref[...] * scale
    ox1 = x1 + offr
    oy1 = y1 + offr
    ox2 = x2 + offr
    oy2 = y2 + offr
    oarea = (ox2 - ox1) * (oy2 - oy1)
    supd_ref[...] = jnp.zeros((1, _PAD), jnp.float32)
    lane128 = jax.lax.broadcasted_iota(jnp.int32, (1, _CH), 1)
    for c in range(_PAD // _CH):
        r = pl.ds(c * _CH, _CH)
        offc = labc_ref[r, 0:1] * scale
        rx1 = bcol_ref[r, 0:1] + offc
        ry1 = bcol_ref[r, 1:2] + offc
        rx2 = bcol_ref[r, 2:3] + offc
        ry2 = bcol_ref[r, 3:4] + offc
        rarea = (rx2 - rx1) * (ry2 - ry1)
        xx1 = jnp.maximum(rx1, ox1)
        yy1 = jnp.maximum(ry1, oy1)
        xx2 = jnp.minimum(rx2, ox2)
        yy2 = jnp.minimum(ry2, oy2)
        inter = (jnp.clip(xx2 - xx1, 0.0) * jnp.clip(yy2 - yy1, 0.0))
        iou = inter / (rarea + oarea - inter + 1e-9)
        sup_ref[...] = (iou > _THR).astype(jnp.float32)

        def _inner(g, s):
            base = pl.multiple_of(g * 8, 8)
            blk = sup_ref[pl.ds(base, 8), r]
            for k in range(8):
                i = base + k
                row_i = blk[k:k + 1, :]
                s_i = jnp.max(jnp.where(lane128 == i, s, 0.0))
                new = jnp.where((row_i > 0.0) & (lane128 > i) & (s_i == 0.0),
                                1.0, 0.0)
                s = jnp.maximum(s, new)
            return s

        s = jax.lax.fori_loop(0, _CH // 8, _inner, supd_ref[0:1, r])
        supd_ref[0:1, r] = s
        kept = 1.0 - s
        counts = jnp.dot(kept, sup_ref[...],
                         preferred_element_type=jnp.float32)
        later = lanes >= (c + 1) * _CH
        supd_ref[...] = jnp.maximum(
            supd_ref[...], jnp.where((counts > 0.0) & later, 1.0, 0.0))
    keep = supd_ref[...] == 0.0
    out_ref[...] = jnp.where(keep, sc_ref[...], 0.0)


def _prep(seg_logits, cls_fg):
    return pl.pallas_call(
        _prep_body,
        grid=(_NF // _BLK,),
        in_specs=[
            pl.BlockSpec((_BLK, _FH * _FW // 128, 128), lambda i: (i, 0, 0)),
            pl.BlockSpec((_BLK, _NC), lambda i: (i, 0)),
        ],
        out_specs=[
            pl.BlockSpec((_BLK, _NC), lambda i: (i, 0)),
            pl.BlockSpec((_BLK, 4), lambda i: (i, 0)),
            pl.BlockSpec((_BLK, 1), lambda i: (i, 0)),
        ],
        out_shape=[
            jax.ShapeDtypeStruct((_NF, _NC), jnp.float32),
            jax.ShapeDtypeStruct((_NF, 4), jnp.float32),
            jax.ShapeDtypeStruct((_NF, 1), jnp.float32),
        ],
        compiler_params=pltpu.CompilerParams(
            dimension_semantics=("parallel",)),
    )(seg_logits, cls_fg)


def _nms(brow, bcol, labr, labc, scr):
    return pl.pallas_call(
        _nms_body,
        out_shape=jax.ShapeDtypeStruct((1, _PAD), jnp.float32),
        scratch_shapes=[
            pltpu.VMEM((_CH, _PAD), jnp.float32),
            pltpu.VMEM((1, _PAD), jnp.float32),
        ],
    )(brow, bcol, labr, labc, scr)


def kernel(cls_logits, seg_logits):
    seg128 = seg_logits.reshape(_NF, _FH * _FW // 128, 128)
    scores, boxes, rowmax = _prep(seg128, cls_logits[:, :-1])
    _, rid = jax.lax.top_k(rowmax[:, 0], _NR)
    rid = jnp.sort(rid)
    subflat = scores[rid].reshape(-1)
    cand_scores, sub_ids = jax.lax.top_k(subflat, _KC)
    cand_feat = rid[sub_ids // _NC]
    cand_labels = sub_ids % _NC
    cboxes = boxes[cand_feat]
    pad = _PAD - _KC
    bcol = jnp.pad(cboxes, ((0, pad), (0, 0)))
    brow = bcol.T
    labf = cand_labels.astype(jnp.float32)
    labr = jnp.pad(labf, (0, pad)).reshape(1, _PAD)
    labc = labr.reshape(_PAD, 1)
    scr = jnp.pad(cand_scores, (0, pad)).reshape(1, _PAD)
    masked = _nms(brow, bcol, labr, labc, scr)[0, :_KC]
    final_scores, top_ids = jax.lax.top_k(masked, _KD)
    final_feat = cand_feat[top_ids]
    final_labels = cand_labels[top_ids]
    final_masks = seg_logits[final_feat] > 0.0
    batch_ids = jnp.zeros((_KD,), jnp.int32)
    return final_labels, final_masks, final_scores, batch_ids


# speedup vs baseline: 1.0868x; 1.0868x over previous
import jax
import jax.numpy as jnp
from jax.experimental import pallas as pl
from jax.experimental.pallas import tpu as pltpu

_NF = 5000
_NC = 80
_FH = 64
_FW = 64
_KC = 1000
_PAD = 1024
_THR = 0.65
_KD = 100
_BLK = 200
_CH = 128
_NR = 1024


def _prep_body(seg_ref, cls_ref, scores_ref, boxes_ref, rowmax_ref):
    m = seg_ref[...] > 0.0
    cols = jnp.any(m, axis=1)
    rows = jnp.any(m, axis=2)
    xs = jax.lax.broadcasted_iota(jnp.int32, cols.shape, 1)
    ys = jax.lax.broadcasted_iota(jnp.int32, rows.shape, 1)
    x1 = jnp.min(jnp.where(cols, xs, _FW), axis=1, keepdims=True)
    x2 = jnp.max(jnp.where(cols, xs + 1, 0), axis=1, keepdims=True)
    y1 = jnp.min(jnp.where(rows, ys, _FH), axis=1, keepdims=True)
    y2 = jnp.max(jnp.where(rows, ys + 1, 0), axis=1, keepdims=True)
    boxes_ref[...] = jnp.concatenate([x1, y1, x2, y2], axis=1).astype(jnp.float32)
    non_empty = jnp.any(cols, axis=1, keepdims=True)
    sc = jnp.where(non_empty, jax.nn.sigmoid(cls_ref[...]), -1.0)
    scores_ref[...] = sc
    rowmax_ref[...] = jnp.max(sc, axis=1, keepdims=True)


def _nms_body(brow_ref, bcol_ref, labr_ref, labc_ref, sc_ref, out_ref,
              sup_ref, supd_ref):
    x1 = brow_ref[0:1, :]
    y1 = brow_ref[1:2, :]
    x2 = brow_ref[2:3, :]
    y2 = brow_ref[3:4, :]
    lanes = jax.lax.broadcasted_iota(jnp.int32, (1, _PAD), 1)
    valid = lanes < _KC
    m4 = jnp.maximum(jnp.maximum(x1, x2), jnp.maximum(y1, y2))
    maxc = jnp.max(jnp.where(valid, m4, -1.0))
    scale = maxc + 1.0
    offr = labr_ref[...] * scale
    ox1 = x1 + offr
    oy1 = y1 + offr
    ox2 = x2 + offr
    oy2 = y2 + offr
    oarea = (ox2 - ox1) * (oy2 - oy1)
    supd_ref[...] = jnp.zeros((1, _PAD), jnp.float32)
    lane128 = jax.lax.broadcasted_iota(jnp.int32, (1, _CH), 1)
    for c in range(_PAD // _CH):
        cs = c * _CH
        r = pl.ds(cs, _CH)
        offc = labc_ref[r, 0:1] * scale
        rx1 = bcol_ref[r, 0:1] + offc
        ry1 = bcol_ref[r, 1:2] + offc
        rx2 = bcol_ref[r, 2:3] + offc
        ry2 = bcol_ref[r, 3:4] + offc
        rarea = (rx2 - rx1) * (ry2 - ry1)
        xx1 = jnp.maximum(rx1, ox1[:, cs:])
        yy1 = jnp.maximum(ry1, oy1[:, cs:])
        xx2 = jnp.minimum(rx2, ox2[:, cs:])
        yy2 = jnp.minimum(ry2, oy2[:, cs:])
        inter = (jnp.clip(xx2 - xx1, 0.0) * jnp.clip(yy2 - yy1, 0.0))
        iou = inter / (rarea + oarea[:, cs:] - inter + 1e-9)
        sup_ref[:, cs:] = (iou > _THR).astype(jnp.float32)

        def _inner(g, s):
            base = pl.multiple_of(g * 8, 8)
            blk = sup_ref[pl.ds(base, 8), r]
            for k in range(8):
                i = base + k
                row_i = blk[k:k + 1, :]
                s_i = jnp.max(jnp.where(lane128 == i, s, 0.0))
                new = jnp.where((row_i > 0.0) & (lane128 > i) & (s_i == 0.0),
                                1.0, 0.0)
                s = jnp.maximum(s, new)
            return s

        s = jax.lax.fori_loop(0, _CH // 8, _inner, supd_ref[0:1, r])
        supd_ref[0:1, r] = s
        if c + 1 < _PAD // _CH:
            ns = cs + _CH
            kept = 1.0 - s
            counts = jnp.dot(kept, sup_ref[:, ns:],
                             preferred_element_type=jnp.float32)
            supd_ref[0:1, ns:] = jnp.maximum(
                supd_ref[0:1, ns:], jnp.where(counts > 0.0, 1.0, 0.0))
    keep = supd_ref[...] == 0.0
    out_ref[...] = jnp.where(keep, sc_ref[...], 0.0)


def _prep(seg_logits, cls_fg):
    return pl.pallas_call(
        _prep_body,
        grid=(_NF // _BLK,),
        in_specs=[
            pl.BlockSpec((_BLK, _FH, _FW), lambda i: (i, 0, 0)),
            pl.BlockSpec((_BLK, _NC), lambda i: (i, 0)),
        ],
        out_specs=[
            pl.BlockSpec((_BLK, _NC), lambda i: (i, 0)),
            pl.BlockSpec((_BLK, 4), lambda i: (i, 0)),
            pl.BlockSpec((_BLK, 1), lambda i: (i, 0)),
        ],
        out_shape=[
            jax.ShapeDtypeStruct((_NF, _NC), jnp.float32),
            jax.ShapeDtypeStruct((_NF, 4), jnp.float32),
            jax.ShapeDtypeStruct((_NF, 1), jnp.float32),
        ],
        compiler_params=pltpu.CompilerParams(
            dimension_semantics=("parallel",)),
    )(seg_logits, cls_fg)


def _nms(brow, bcol, labr, labc, scr):
    return pl.pallas_call(
        _nms_body,
        out_shape=jax.ShapeDtypeStruct((1, _PAD), jnp.float32),
        scratch_shapes=[
            pltpu.VMEM((_CH, _PAD), jnp.float32),
            pltpu.VMEM((1, _PAD), jnp.float32),
        ],
    )(brow, bcol, labr, labc, scr)


def kernel(cls_logits, seg_logits):
    scores, boxes, rowmax = _prep(seg_logits, cls_logits[:, :-1])
    _, rid = jax.lax.top_k(rowmax[:, 0], _NR)
    rid = jnp.sort(rid)
    subflat = scores[rid].reshape(-1)
    cand_scores, sub_ids = jax.lax.top_k(subflat, _KC)
    cand_feat = rid[sub_ids // _NC]
    cand_labels = sub_ids % _NC
    cboxes = boxes[cand_feat]
    pad = _PAD - _KC
    bcol = jnp.pad(cboxes, ((0, pad), (0, 0)))
    brow = bcol.T
    labf = cand_labels.astype(jnp.float32)
    labr = jnp.pad(labf, (0, pad)).reshape(1, _PAD)
    labc = labr.reshape(_PAD, 1)
    scr = jnp.pad(cand_scores, (0, pad)).reshape(1, _PAD)
    masked = _nms(brow, bcol, labr, labc, scr)[0, :_KC]
    final_scores, top_ids = jax.lax.top_k(masked, _KD)
    final_feat = cand_feat[top_ids]
    final_labels = cand_labels[top_ids]
    final_masks = seg_logits[final_feat] > 0.0
    batch_ids = jnp.zeros((_KD,), jnp.int32)
    return final_labels, final_masks, final_scores, batch_ids
